# baseline (speedup 1.0000x reference)
import functools

import jax
import jax.numpy as jnp
from jax import lax
from jax.experimental import pallas as pl
from jax.experimental.pallas import tpu as pltpu
from jax.experimental.pallas import tpu_sc as plsc

BATCH = 4096
SEQ = 200
EMBED_DIM = 64
NUM_CLASS = 4

NUM_WORKERS = 32
ROWS_PER_W = BATCH // NUM_WORKERS
HALF = SEQ // 2


def _pooling_kernel(text_hbm, table_hbm, pooled_hbm, idx_v, rows_v, pooled_v,
                    sem):
  wid = lax.axis_index("s") * 2 + lax.axis_index("c")
  base = wid * ROWS_PER_W

  pltpu.sync_copy(text_hbm.at[pl.ds(base, ROWS_PER_W)], idx_v)

  @pl.loop(0, ROWS_PER_W)
  def _row(i):
    pltpu.async_copy(table_hbm.at[idx_v.at[i, 0]], rows_v.at[0], sem).wait()
    pltpu.async_copy(table_hbm.at[idx_v.at[i, 1]], rows_v.at[1], sem).wait()

    def body(t, accs):
      out = []
      for k in range(4):
        a = accs[k]
        a = a + rows_v[0, t, pl.ds(k * 16, 16)]
        a = a + rows_v[1, t, pl.ds(k * 16, 16)]
        out.append(a)
      return tuple(out)

    zero = jnp.zeros((16,), jnp.float32)
    accs = lax.fori_loop(0, HALF, body, (zero, zero, zero, zero))
    for k in range(4):
      pooled_v[i, pl.ds(k * 16, 16)] = accs[k]

  pltpu.sync_copy(pooled_v, pooled_hbm.at[pl.ds(base, ROWS_PER_W)])


@jax.jit
def _pooled_sum(text, table):
  mesh = plsc.VectorSubcoreMesh(core_axis_name="c", subcore_axis_name="s")
  f = pl.kernel(
      _pooling_kernel,
      out_type=jax.ShapeDtypeStruct((BATCH, EMBED_DIM), jnp.float32),
      mesh=mesh,
      compiler_params=pltpu.CompilerParams(use_tc_tiling_on_sc=False),
      scratch_types=[
          pltpu.VMEM((ROWS_PER_W, 2, HALF), jnp.int32),
          pltpu.VMEM((2, HALF, EMBED_DIM), jnp.float32),
          pltpu.VMEM((ROWS_PER_W, EMBED_DIM), jnp.float32),
          pltpu.SemaphoreType.DMA,
      ],
  )
  return f(text.reshape(BATCH, 2, HALF), table)


def _linear_body(p_ref, w_ref, b_ref, o_ref):
  p = p_ref[...] * (1.0 / SEQ)
  o_ref[...] = jnp.dot(p, w_ref[...].T,
                       preferred_element_type=jnp.float32) + b_ref[...]


@jax.jit
def _linear(pooled_sum, W, b):
  return pl.pallas_call(
      _linear_body,
      out_shape=jax.ShapeDtypeStruct((BATCH, NUM_CLASS), jnp.float32),
  )(pooled_sum, W, b.reshape(1, NUM_CLASS))


def kernel(text, lengths, table, W, b):
  del lengths
  pooled_sum = _pooled_sum(text, table)
  return _linear(pooled_sum, W, b)


# speedup vs baseline: 1.2971x; 1.2971x over previous
import functools

import jax
import jax.numpy as jnp
from jax import lax
from jax.experimental import pallas as pl
from jax.experimental.pallas import tpu as pltpu
from jax.experimental.pallas import tpu_sc as plsc

BATCH = 4096
SEQ = 200
EMBED_DIM = 64
NUM_CLASS = 4

NUM_WORKERS = 32
ROWS_PER_W = BATCH // NUM_WORKERS
HALF = SEQ // 2


NBUF = 4


def _pooling_kernel(text_hbm, table_hbm, pooled_hbm, idx_v, rows_v, pooled_v,
                    sems):
  wid = lax.axis_index("s") * 2 + lax.axis_index("c")
  base = wid * ROWS_PER_W

  pltpu.sync_copy(text_hbm.at[pl.ds(base, ROWS_PER_W)], idx_v)

  def issue(row, slot):
    pltpu.async_copy(table_hbm.at[idx_v.at[row, 0]], rows_v.at[slot, 0],
                     sems.at[slot])
    pltpu.async_copy(table_hbm.at[idx_v.at[row, 1]], rows_v.at[slot, 1],
                     sems.at[slot])

  def drain(slot):
    for h in range(2):
      pltpu.make_async_copy(pooled_hbm.at[pl.ds(0, HALF)],
                            rows_v.at[slot, h], sems.at[slot]).wait()

  for s in range(NBUF):
    issue(s, s)

  @pl.loop(0, ROWS_PER_W, step=NBUF)
  def _rows(i0):
    for s in range(NBUF):
      i = i0 + s
      drain(s)

      def body(t, accs, s=s):
        out = []
        for k in range(4):
          a = accs[k]
          a = a + rows_v[s, 0, t, pl.ds(k * 16, 16)]
          a = a + rows_v[s, 1, t, pl.ds(k * 16, 16)]
          out.append(a)
        return tuple(out)

      zero = jnp.zeros((16,), jnp.float32)
      accs = lax.fori_loop(0, HALF, body, (zero, zero, zero, zero),
                           unroll=2)
      for k in range(4):
        pooled_v[i, pl.ds(k * 16, 16)] = accs[k]

      @pl.when(i + NBUF < ROWS_PER_W)
      def _():
        issue(i + NBUF, s)

  pltpu.sync_copy(pooled_v, pooled_hbm.at[pl.ds(base, ROWS_PER_W)])


@jax.jit
def _pooled_sum(text, table):
  mesh = plsc.VectorSubcoreMesh(core_axis_name="c", subcore_axis_name="s")
  f = pl.kernel(
      _pooling_kernel,
      out_type=jax.ShapeDtypeStruct((BATCH, EMBED_DIM), jnp.float32),
      mesh=mesh,
      compiler_params=pltpu.CompilerParams(use_tc_tiling_on_sc=False),
      scratch_types=[
          pltpu.VMEM((ROWS_PER_W, 2, HALF), jnp.int32),
          pltpu.VMEM((NBUF, 2, HALF, EMBED_DIM), jnp.float32),
          pltpu.VMEM((ROWS_PER_W, EMBED_DIM), jnp.float32),
          pltpu.SemaphoreType.DMA((NBUF,)),
      ],
  )
  return f(text.reshape(BATCH, 2, HALF), table)


def _linear_body(p_ref, w_ref, b_ref, o_ref):
  p = p_ref[...] * (1.0 / SEQ)
  o_ref[...] = jnp.dot(p, w_ref[...].T,
                       preferred_element_type=jnp.float32) + b_ref[...]


@jax.jit
def _linear(pooled_sum, W, b):
  return pl.pallas_call(
      _linear_body,
      out_shape=jax.ShapeDtypeStruct((BATCH, NUM_CLASS), jnp.float32),
  )(pooled_sum, W, b.reshape(1, NUM_CLASS))


def kernel(text, lengths, table, W, b):
  del lengths
  pooled_sum = _pooled_sum(text, table)
  return _linear(pooled_sum, W, b)


# speedup vs baseline: 1.3101x; 1.0101x over previous
import functools

import jax
import jax.numpy as jnp
from jax import lax
from jax.experimental import pallas as pl
from jax.experimental.pallas import tpu as pltpu
from jax.experimental.pallas import tpu_sc as plsc

BATCH = 4096
SEQ = 200
EMBED_DIM = 64
NUM_CLASS = 4

NUM_WORKERS = 32
ROWS_PER_W = BATCH // NUM_WORKERS
SPLIT = (96, 104)


NBUF = 4


def _pooling_kernel(text_hbm, table_hbm, pooled_hbm, idx_v, rows_v, pooled_v,
                    sems):
  wid = lax.axis_index("s") * 2 + lax.axis_index("c")
  base = wid * ROWS_PER_W

  pltpu.sync_copy(text_hbm.at[pl.ds(base, ROWS_PER_W)], idx_v)

  def issue(row, slot):
    off = 0
    for n in SPLIT:
      pltpu.async_copy(table_hbm.at[idx_v.at[row, pl.ds(off, n)]],
                       rows_v.at[slot, pl.ds(off, n)], sems.at[slot])
      off += n

  def drain(slot):
    pltpu.make_async_copy(pooled_hbm.at[pl.ds(0, SEQ)],
                          rows_v.at[slot], sems.at[slot]).wait()

  for s in range(NBUF):
    issue(s, s)

  @pl.loop(0, ROWS_PER_W, step=NBUF)
  def _rows(i0):
    for s in range(NBUF):
      i = i0 + s
      drain(s)

      def body(t, accs, s=s):
        out = []
        for k in range(4):
          a = accs[k]
          a = a + rows_v[s, t, pl.ds(k * 16, 16)]
          out.append(a)
        return tuple(out)

      zero = jnp.zeros((16,), jnp.float32)
      accs = lax.fori_loop(0, SEQ, body, (zero, zero, zero, zero),
                           unroll=4)
      for k in range(4):
        pooled_v[i, pl.ds(k * 16, 16)] = accs[k]

      @pl.when(i + NBUF < ROWS_PER_W)
      def _():
        issue(i + NBUF, s)

  pltpu.sync_copy(pooled_v, pooled_hbm.at[pl.ds(base, ROWS_PER_W)])


@jax.jit
def _pooled_sum(text, table):
  mesh = plsc.VectorSubcoreMesh(core_axis_name="c", subcore_axis_name="s")
  f = pl.kernel(
      _pooling_kernel,
      out_type=jax.ShapeDtypeStruct((BATCH, EMBED_DIM), jnp.float32),
      mesh=mesh,
      compiler_params=pltpu.CompilerParams(use_tc_tiling_on_sc=False),
      scratch_types=[
          pltpu.VMEM((ROWS_PER_W, SEQ), jnp.int32),
          pltpu.VMEM((NBUF, SEQ, EMBED_DIM), jnp.float32),
          pltpu.VMEM((ROWS_PER_W, EMBED_DIM), jnp.float32),
          pltpu.SemaphoreType.DMA((NBUF,)),
      ],
  )
  return f(text, table)


def _linear_body(p_ref, w_ref, b_ref, o_ref):
  p = p_ref[...] * (1.0 / SEQ)
  o_ref[...] = jnp.dot(p, w_ref[...].T,
                       preferred_element_type=jnp.float32) + b_ref[...]


@jax.jit
def _linear(pooled_sum, W, b):
  return pl.pallas_call(
      _linear_body,
      out_shape=jax.ShapeDtypeStruct((BATCH, NUM_CLASS), jnp.float32),
  )(pooled_sum, W, b.reshape(1, NUM_CLASS))


def kernel(text, lengths, table, W, b):
  del lengths
  pooled_sum = _pooled_sum(text, table)
  return _linear(pooled_sum, W, b)


# speedup vs baseline: 2.9998x; 2.2897x over previous
import functools

import jax
import jax.numpy as jnp
from jax import lax
from jax.experimental import pallas as pl
from jax.experimental.pallas import tpu as pltpu
from jax.experimental.pallas import tpu_sc as plsc

BATCH = 4096
SEQ = 200
EMBED_DIM = 64
NUM_CLASS = 4
VOCAB = 1000000
GW = 8

GBLK = 4096
VPAD = 245 * GBLK

NUM_WORKERS = 32
ROWS_PER_W = BATCH // NUM_WORKERS
SPLIT = (96, 104)
NBUF = 4

V_PER_W = VPAD // NUM_WORKERS
TCH = 4480
NTCH = V_PER_W // TCH


def _g_body(w_ref, tT_ref, o_ref):
  o_ref[...] = lax.dot_general(w_ref[...], tT_ref[...],
                               (((1,), (0,)), ((), ())),
                               preferred_element_type=jnp.float32)


@jax.jit
def _class_table(table, W):
  return pl.pallas_call(
      _g_body,
      grid=(VPAD // GBLK,),
      in_specs=[
          pl.BlockSpec((NUM_CLASS, EMBED_DIM), lambda i: (0, 0)),
          pl.BlockSpec((EMBED_DIM, GBLK), lambda i: (0, i)),
      ],
      out_specs=pl.BlockSpec((NUM_CLASS, GBLK), lambda i: (0, i)),
      out_shape=jax.ShapeDtypeStruct((NUM_CLASS, VPAD), jnp.float32),
  )(W, table.T)


def _transpose_kernel(gt_hbm, flat_hbm, buf_v, out_v):
  wid = lax.axis_index("s") * 2 + lax.axis_index("c")
  v0 = wid * V_PER_W
  lane = lax.iota(jnp.int32, 16)
  lane8 = lane * GW

  @pl.loop(0, NTCH)
  def _chunk(k):
    off = v0 + k * TCH
    pltpu.sync_copy(gt_hbm.at[:, pl.ds(off, TCH)], buf_v)

    @pl.loop(0, TCH // 16)
    def _group(g):
      for c in range(NUM_CLASS):
        x = buf_v[c, pl.ds(g * 16, 16)]
        plsc.store_scatter(out_v, [lane8 + (g * 128 + c)], x)

    pltpu.sync_copy(out_v, flat_hbm.at[pl.ds(off * GW, TCH * GW)])


@jax.jit
def _flat_class_table(g_t):
  mesh = plsc.VectorSubcoreMesh(core_axis_name="c", subcore_axis_name="s")
  f = pl.kernel(
      _transpose_kernel,
      out_type=jax.ShapeDtypeStruct((VPAD * GW,), jnp.float32),
      mesh=mesh,
      compiler_params=pltpu.CompilerParams(use_tc_tiling_on_sc=True,
                                           needs_layout_passes=False),
      scratch_types=[
          pltpu.VMEM((NUM_CLASS, TCH), jnp.float32),
          pltpu.VMEM((TCH * GW,), jnp.float32),
      ],
  )
  return f(g_t)


def _pooling_kernel(text_hbm, g_hbm, pooled_hbm, idx_v, pooled_v, sems,
                    *rows_bufs):
  wid = lax.axis_index("s") * 2 + lax.axis_index("c")
  base = wid * ROWS_PER_W

  pltpu.sync_copy(text_hbm.at[pl.ds(base, ROWS_PER_W)], idx_v)

  def issue(row, slot):
    off = 0
    for n in SPLIT:
      pltpu.async_copy(g_hbm.at[idx_v.at[row, pl.ds(off, n)]],
                       rows_bufs[slot].at[pl.ds(off, n)], sems.at[slot])
      off += n

  def drain(slot):
    pltpu.make_async_copy(g_hbm.at[pl.ds(0, SEQ)], rows_bufs[slot],
                          sems.at[slot]).wait()

  for s in range(NBUF):
    issue(s, s)

  lane = lax.iota(jnp.int32, 16)
  qrow = lax.shift_right_logical(lane, 2)
  qcol = lax.bitwise_and(lane, 3)

  @pl.loop(0, ROWS_PER_W, step=NBUF)
  def _rows(i0):
    for s in range(NBUF):
      i = i0 + s
      drain(s)

      def body(u, acc, s=s):
        x = plsc.load_gather(rows_bufs[s], [u * 4 + qrow, qcol])
        return acc + x

      acc = lax.fori_loop(0, SEQ // 4, body, jnp.zeros((16,), jnp.float32),
                          unroll=5)
      pooled_v[i] = acc

      @pl.when(i + NBUF < ROWS_PER_W)
      def _():
        issue(i + NBUF, s)

  pltpu.sync_copy(pooled_v, pooled_hbm.at[pl.ds(base, ROWS_PER_W)])


@jax.jit
def _pooled_sum(text, g_flat):
  mesh = plsc.VectorSubcoreMesh(core_axis_name="c", subcore_axis_name="s")
  f = pl.kernel(
      _pooling_kernel,
      out_type=jax.ShapeDtypeStruct((BATCH, 16), jnp.float32),
      mesh=mesh,
      compiler_params=pltpu.CompilerParams(use_tc_tiling_on_sc=False,
                                           needs_layout_passes=False),
      scratch_types=[
          pltpu.VMEM((ROWS_PER_W, SEQ), jnp.int32),
          pltpu.VMEM((ROWS_PER_W, 16), jnp.float32),
          pltpu.SemaphoreType.DMA((NBUF,)),
      ] + [
          pltpu.VMEM((SEQ, GW), jnp.float32) for _ in range(NBUF)
      ],
  )
  return f(text, g_flat.reshape(VPAD, GW))


def _linear_body(p_ref, m_ref, b_ref, o_ref):
  o_ref[...] = jnp.dot(p_ref[...], m_ref[...],
                       preferred_element_type=jnp.float32) * (1.0 / SEQ) \
      + b_ref[...]


@jax.jit
def _linear(pooled16, b):
  fold = jnp.tile(jnp.eye(NUM_CLASS, dtype=jnp.float32), (4, 1))
  return pl.pallas_call(
      _linear_body,
      out_shape=jax.ShapeDtypeStruct((BATCH, NUM_CLASS), jnp.float32),
  )(pooled16, fold, b.reshape(1, NUM_CLASS))


def kernel(text, lengths, table, W, b):
  del lengths
  g_t = _class_table(table, W)
  g_flat = _flat_class_table(g_t)
  pooled16 = _pooled_sum(text, g_flat)
  return _linear(pooled16, b)
